# baseline (speedup 1.0000x reference)
import functools

import jax
import jax.numpy as jnp
from jax import lax
from jax.experimental import pallas as pl
from jax.experimental.pallas import tpu as pltpu
from jax.experimental.pallas import tpu_sc as plsc

D = 64
B = 16384

_NC = 2
_NS = 16
_NW = _NC * _NS
_BPW = B // _NW
_CHUNK = 128
_NCHUNK = _BPW // _CHUNK

_mesh = plsc.VectorSubcoreMesh(core_axis_name="c", subcore_axis_name="s")


@functools.partial(
    pl.kernel,
    mesh=_mesh,
    compiler_params=pltpu.CompilerParams(use_tc_tiling_on_sc=False),
    out_type=jax.ShapeDtypeStruct((B,), jnp.float32),
    scratch_types=[
        pltpu.VMEM((_BPW,), jnp.int32),
        pltpu.VMEM((_BPW, D), jnp.float32),
        pltpu.VMEM((D,), jnp.float32),
        pltpu.VMEM((_BPW,), jnp.float32),
        pltpu.SemaphoreType.DMA,
    ],
)
def _mf_kernel(items_hbm, u_hbm, table_hbm, out_hbm, idx_v, rows_v, u_v, out_v, sem):
    wid = lax.axis_index("s") * _NC + lax.axis_index("c")
    base = wid * _BPW
    pltpu.sync_copy(items_hbm.at[pl.ds(base, _BPW)], idx_v)
    pltpu.sync_copy(u_hbm, u_v)
    copies = [
        pltpu.async_copy(
            table_hbm.at[idx_v.at[pl.ds(j * _CHUNK, _CHUNK)]],
            rows_v.at[pl.ds(j * _CHUNK, _CHUNK)],
            sem,
        )
        for j in range(_NCHUNK)
    ]
    for c in copies:
        c.wait()

    u0 = u_v[pl.ds(0, 16)]
    u1 = u_v[pl.ds(16, 16)]
    u2 = u_v[pl.ds(32, 16)]
    u3 = u_v[pl.ds(48, 16)]

    lanes = lax.iota(jnp.int32, 16)
    shuf = [lanes ^ k for k in (1, 2, 4, 8)]

    def body(g, carry):
        base_i = g * 16
        res = jnp.zeros((16,), jnp.float32)
        for l in range(16):
            i = base_i + l
            t = (rows_v[i, pl.ds(0, 16)] * u0
                 + rows_v[i, pl.ds(16, 16)] * u1
                 + rows_v[i, pl.ds(32, 16)] * u2
                 + rows_v[i, pl.ds(48, 16)] * u3)
            for s in shuf:
                t = t + t.at[s].get(mode="promise_in_bounds")
            res = jnp.where(lanes == l, t, res)
        out_v[pl.ds(base_i, 16)] = res
        return carry

    lax.fori_loop(0, _BPW // 16, body, 0)
    pltpu.sync_copy(out_v, out_hbm.at[pl.ds(base, _BPW)])


def kernel(users, items, user_factors, item_factors):
    del users
    u = user_factors.reshape((D,))
    return _mf_kernel(items, u, item_factors)


# speedup vs baseline: 2.9423x; 2.9423x over previous
import functools

import jax
import jax.numpy as jnp
from jax import lax
from jax.experimental import pallas as pl
from jax.experimental.pallas import tpu as pltpu
from jax.experimental.pallas import tpu_sc as plsc

D = 64
B = 16384

_NC = 2
_NS = 16
_NW = _NC * _NS
_BPW = B // _NW
_NBUF = 8
_NIT = _BPW // _NBUF

_mesh = plsc.VectorSubcoreMesh(core_axis_name="c", subcore_axis_name="s")


@functools.partial(
    pl.kernel,
    mesh=_mesh,
    compiler_params=pltpu.CompilerParams(needs_layout_passes=False),
    out_type=jax.ShapeDtypeStruct((B,), jnp.float32),
    scratch_types=[
        pltpu.VMEM((_BPW + 16,), jnp.int32),
        pltpu.VMEM((_NBUF, D, 128), jnp.float32),
        pltpu.VMEM((D,), jnp.float32),
        pltpu.VMEM((_BPW,), jnp.float32),
    ]
    + [pltpu.SemaphoreType.DMA] * _NBUF,
)
def _mf_kernel(items_hbm, u_hbm, tt_hbm, out_hbm, idx_v, bufs, u_v, out_v,
               *sems):
    wid = lax.axis_index("s") * _NC + lax.axis_index("c")
    base = wid * _BPW
    pltpu.sync_copy(items_hbm.at[pl.ds(base, _BPW)], idx_v.at[pl.ds(0, _BPW)])
    pltpu.sync_copy(u_hbm, u_v)
    lanes = lax.iota(jnp.int32, 16)
    u_vecs = [u_v[pl.ds(q * 16, 16)] for q in range(D // 16)]

    def issue(start_scalar, slot, sem):
        start = pl.multiple_of(start_scalar, 128)
        pltpu.async_copy(tt_hbm.at[:, pl.ds(start, 128)], bufs.at[slot], sem)

    starts0 = lax.bitwise_and(idx_v[pl.ds(0, 16)], jnp.int32(~127))
    for j in range(_NBUF):
        issue(starts0[j], j, sems[j])

    def body(t, carry):
        iv = idx_v[pl.ds(t * _NBUF, 16)]
        starts = lax.bitwise_and(iv, jnp.int32(~127))
        loffs = lax.bitwise_and(iv, jnp.int32(112))
        l15 = lax.bitwise_and(iv, jnp.int32(15))
        res = jnp.zeros((16,), jnp.float32)
        for j in range(_NBUF):
            pltpu.make_async_copy(
                tt_hbm.at[:, pl.ds(0, 128)], bufs.at[j], sems[j]
            ).wait()
            loff = loffs[j]
            acc = jnp.zeros((16,), jnp.float32)
            for f in range(D):
                acc = acc + (bufs[j, f, pl.ds(loff, 16)]
                             * u_vecs[f // 16][f % 16])

            @pl.when(t + 1 < _NIT)
            def _():
                issue(starts[j + 8], j, sems[j])

            pick = acc.at[jnp.full((16,), l15[j], jnp.int32)].get(
                mode="promise_in_bounds")
            res = jnp.where(lanes == j, pick, res)
        plsc.store_scatter(
            out_v,
            [t * _NBUF + lax.bitwise_and(lanes, jnp.int32(7))],
            res,
            mask=lanes < _NBUF,
        )
        return carry

    lax.fori_loop(0, _NIT, body, 0)
    pltpu.sync_copy(out_v, out_hbm.at[pl.ds(base, _BPW)])


def kernel(users, items, user_factors, item_factors):
    del users
    u = user_factors.reshape((D,))
    return _mf_kernel(items, u, item_factors.T)
